# baseline (speedup 1.0000x reference)
import functools

import jax
import jax.numpy as jnp
from jax.experimental import pallas as pl
from jax.experimental.pallas import tpu as pltpu

_KNN_K = 32
_NUM_CLUSTERS = 16
_KMEANS_ITERS = 10
_BISECT_ITERS = 14


def _cluster_body(x_ref, xt_ref, factor_ref, ct_ref, c_ref, csum_ref):
    x = x_ref[...]
    xt = xt_ref[...]
    n = x.shape[0]
    k = _NUM_CLUSTERS
    centers = x[:k, :]

    def body(_, centers):
        sq = centers * centers
        cnorm = jnp.sum(sq, axis=1, keepdims=True)
        prod = jnp.dot(centers, xt, preferred_element_type=jnp.float32)
        score = cnorm - 2.0 * prod
        mn = jnp.min(score, axis=0, keepdims=True)
        ids = jax.lax.broadcasted_iota(jnp.int32, (k, n), 0)
        first = jnp.min(jnp.where(score == mn, ids, k), axis=0, keepdims=True)
        onehot = (ids == first).astype(jnp.float32)
        sums = jnp.dot(onehot, x, preferred_element_type=jnp.float32)
        counts = jnp.sum(onehot, axis=1, keepdims=True)
        return sums / jnp.maximum(counts, 1.0)

    centers = jax.lax.fori_loop(0, _KMEANS_ITERS, body, centers)

    xnorm = jnp.sum(xt * xt, axis=0, keepdims=True)
    cnorm = jnp.sum(centers * centers, axis=1, keepdims=True)
    prod = jnp.dot(centers, xt, preferred_element_type=jnp.float32)
    d2 = xnorm + cnorm - 2.0 * prod
    e = jnp.exp(-d2 * factor_ref[0, 0])
    colsum = jnp.sum(e, axis=0, keepdims=True)
    ct = e / (colsum + 1e-10)
    ct_ref[...] = ct
    eye = (jax.lax.broadcasted_iota(jnp.int32, (k, k), 0)
           == jax.lax.broadcasted_iota(jnp.int32, (k, k), 1)).astype(jnp.float32)
    c = jax.lax.dot_general(ct, eye, (((0,), (0,)), ((), ())),
                            preferred_element_type=jnp.float32)
    c_ref[...] = c
    csum_ref[...] = jnp.dot(jnp.ones((1, n), jnp.float32), c,
                            preferred_element_type=jnp.float32)


def _main_body(x_ref, xt_ref, ct_ref, cblk_ref, csum_ref,
               loss_ref, accm_ref, accq_ref, accd_ref, accs_ref):
    i = pl.program_id(0)
    nblocks = pl.num_programs(0)
    br = x_ref.shape[0]
    n = xt_ref.shape[1]

    @pl.when(i == 0)
    def _init():
        accm_ref[...] = jnp.zeros_like(accm_ref)
        accq_ref[...] = jnp.zeros_like(accq_ref)
        accd_ref[...] = jnp.zeros_like(accd_ref)
        accs_ref[...] = jnp.zeros_like(accs_ref)

    s = jnp.dot(x_ref[...].astype(jnp.bfloat16), xt_ref[...],
                preferred_element_type=jnp.float32)

    gm = s[:, 0:128]
    for c in range(1, 32):
        gm = jnp.maximum(gm, s[:, c * 128:(c + 1) * 128])
    lo = jnp.min(gm, axis=1, keepdims=True)
    hi = jnp.max(gm, axis=1, keepdims=True) + 1.0

    def bisect(_, carry):
        lo, hi = carry
        mid = 0.5 * (lo + hi)
        cnt = jnp.sum(jnp.where(s >= mid, 1.0, 0.0), axis=1, keepdims=True)
        ge = cnt >= _KNN_K
        return jnp.where(ge, mid, lo), jnp.where(ge, hi, mid)

    lo, hi = jax.lax.fori_loop(0, _BISECT_ITERS, bisect, (lo, hi))
    thresh = lo

    w = jnp.where(s >= thresh, 0.5 * jnp.tanh(0.5 * s), 0.0)
    wsum = jnp.sum(w, axis=1, keepdims=True)
    arow = 0.5 * n + wsum
    r = 1.0 / (arow + 1e-8)

    v = jax.lax.dot_general(w, ct_ref[...], (((1,), (1,)), ((), ())),
                            preferred_element_type=jnp.float32)

    sdiag = jnp.sum(x_ref[...] * x_ref[...], axis=1, keepdims=True)
    wdiag = jnp.where(sdiag >= thresh, 0.5 * jnp.tanh(0.5 * sdiag), 0.0)
    d = r * (0.5 + wdiag)

    cblk = cblk_ref[...]
    m_blk = r * (0.5 * csum_ref[...] + v) - d * cblk

    accm_ref[...] += jnp.sum(m_blk, axis=0, keepdims=True)
    accq_ref[...] += jnp.sum(cblk * m_blk, axis=0, keepdims=True)
    accd_ref[...] += jnp.sum(d, axis=0, keepdims=True)
    accs_ref[...] += jnp.sum(arow * r, axis=0, keepdims=True)

    @pl.when(i == nblocks - 1)
    def _fini():
        deno = 1.0 / (accs_ref[...] - accd_ref[...] + 1e-10)
        m = accm_ref[...] * deno
        q = accq_ref[...] * deno
        encolen = jnp.log2(m + 1e-20)
        loss_ref[...] = -jnp.sum((m - q) * encolen, axis=1, keepdims=True)


@jax.jit
def kernel(logits, sigma):
    n, dmod = logits.shape
    k = _NUM_CLUSTERS
    xt = logits.T
    sigma_f = jnp.asarray(sigma, dtype=jnp.float32)
    factor = (1.0 / (2.0 * sigma_f ** 2)).reshape(1, 1)

    ct, c, csum = pl.pallas_call(
        _cluster_body,
        out_shape=(
            jax.ShapeDtypeStruct((k, n), jnp.float32),
            jax.ShapeDtypeStruct((n, k), jnp.float32),
            jax.ShapeDtypeStruct((1, k), jnp.float32),
        ),
    )(logits, xt, factor)

    br = 1024
    nblocks = n // br
    xt_bf = xt.astype(jnp.bfloat16)
    loss = pl.pallas_call(
        _main_body,
        grid=(nblocks,),
        in_specs=[
            pl.BlockSpec((br, dmod), lambda i: (i, 0)),
            pl.BlockSpec((dmod, n), lambda i: (0, 0)),
            pl.BlockSpec((k, n), lambda i: (0, 0)),
            pl.BlockSpec((br, k), lambda i: (i, 0)),
            pl.BlockSpec((1, k), lambda i: (0, 0)),
        ],
        out_specs=pl.BlockSpec((1, 1), lambda i: (0, 0)),
        out_shape=jax.ShapeDtypeStruct((1, 1), jnp.float32),
        scratch_shapes=[
            pltpu.VMEM((1, k), jnp.float32),
            pltpu.VMEM((1, k), jnp.float32),
            pltpu.VMEM((1, 1), jnp.float32),
            pltpu.VMEM((1, 1), jnp.float32),
        ],
    )(logits, xt_bf, ct, c, csum)
    return loss[0, 0]


# speedup vs baseline: 1.0124x; 1.0124x over previous
import functools

import jax
import jax.numpy as jnp
from jax.experimental import pallas as pl
from jax.experimental.pallas import tpu as pltpu

_KNN_K = 32
_NUM_CLUSTERS = 16
_KMEANS_ITERS = 10
_BISECT_ITERS = 14


def _cluster_body(x_ref, xt_ref, factor_ref, ct_ref, c_ref, csum_ref):
    x = x_ref[...]
    xt = xt_ref[...]
    n = x.shape[0]
    k = _NUM_CLUSTERS
    centers = x[:k, :]

    def body(_, centers):
        sq = centers * centers
        cnorm = jnp.sum(sq, axis=1, keepdims=True)
        prod = jnp.dot(centers, xt, preferred_element_type=jnp.float32)
        score = cnorm - 2.0 * prod
        mn = jnp.min(score, axis=0, keepdims=True)
        ids = jax.lax.broadcasted_iota(jnp.int32, (k, n), 0)
        first = jnp.min(jnp.where(score == mn, ids, k), axis=0, keepdims=True)
        onehot = (ids == first).astype(jnp.float32)
        sums = jnp.dot(onehot, x, preferred_element_type=jnp.float32)
        counts = jnp.sum(onehot, axis=1, keepdims=True)
        return sums / jnp.maximum(counts, 1.0)

    centers = jax.lax.fori_loop(0, _KMEANS_ITERS, body, centers)

    xnorm = jnp.sum(xt * xt, axis=0, keepdims=True)
    cnorm = jnp.sum(centers * centers, axis=1, keepdims=True)
    prod = jnp.dot(centers, xt, preferred_element_type=jnp.float32)
    d2 = xnorm + cnorm - 2.0 * prod
    e = jnp.exp(-d2 * factor_ref[0, 0])
    colsum = jnp.sum(e, axis=0, keepdims=True)
    ct = e / (colsum + 1e-10)
    ct_ref[...] = ct
    eye = (jax.lax.broadcasted_iota(jnp.int32, (k, k), 0)
           == jax.lax.broadcasted_iota(jnp.int32, (k, k), 1)).astype(jnp.float32)
    c = jax.lax.dot_general(ct, eye, (((0,), (0,)), ((), ())),
                            preferred_element_type=jnp.float32)
    c_ref[...] = c
    csum_ref[...] = jnp.dot(jnp.ones((1, n), jnp.float32), c,
                            preferred_element_type=jnp.float32)


def _main_body(x_ref, xt_ref, ct_ref, cblk_ref, csum_ref,
               loss_ref, accm_ref, accq_ref, accd_ref, accs_ref):
    i = pl.program_id(0)
    nblocks = pl.num_programs(0)
    br = x_ref.shape[0]
    n = xt_ref.shape[1]

    @pl.when(i == 0)
    def _init():
        accm_ref[...] = jnp.zeros_like(accm_ref)
        accq_ref[...] = jnp.zeros_like(accq_ref)
        accd_ref[...] = jnp.zeros_like(accd_ref)
        accs_ref[...] = jnp.zeros_like(accs_ref)

    s = jnp.dot(x_ref[...], xt_ref[...], preferred_element_type=jnp.float32)

    gm = s[:, 0:128]
    for c in range(1, 32):
        gm = jnp.maximum(gm, s[:, c * 128:(c + 1) * 128])
    lo = jnp.min(gm, axis=1, keepdims=True)
    hi = jnp.max(gm, axis=1, keepdims=True) + 1.0

    def bisect(_, carry):
        lo, hi = carry
        mid = 0.5 * (lo + hi)
        cnt = jnp.sum(jnp.where(s >= mid, 1.0, 0.0), axis=1, keepdims=True)
        ge = cnt >= _KNN_K
        return jnp.where(ge, mid, lo), jnp.where(ge, hi, mid)

    lo, hi = jax.lax.fori_loop(0, _BISECT_ITERS, bisect, (lo, hi))
    thresh = lo

    w = jnp.where(s >= thresh, 0.5 * jnp.tanh(0.5 * s), 0.0)
    wsum = jnp.sum(w, axis=1, keepdims=True)
    arow = 0.5 * n + wsum
    r = 1.0 / (arow + 1e-8)

    v = jax.lax.dot_general(w, ct_ref[...], (((1,), (1,)), ((), ())),
                            preferred_element_type=jnp.float32)

    sdiag = jnp.sum(x_ref[...] * x_ref[...], axis=1, keepdims=True)
    wdiag = jnp.where(sdiag >= thresh, 0.5 * jnp.tanh(0.5 * sdiag), 0.0)
    d = r * (0.5 + wdiag)

    cblk = cblk_ref[...]
    m_blk = r * (0.5 * csum_ref[...] + v) - d * cblk

    accm_ref[...] += jnp.sum(m_blk, axis=0, keepdims=True)
    accq_ref[...] += jnp.sum(cblk * m_blk, axis=0, keepdims=True)
    accd_ref[...] += jnp.sum(d, axis=0, keepdims=True)
    accs_ref[...] += jnp.sum(arow * r, axis=0, keepdims=True)

    @pl.when(i == nblocks - 1)
    def _fini():
        deno = 1.0 / (accs_ref[...] - accd_ref[...] + 1e-10)
        m = accm_ref[...] * deno
        q = accq_ref[...] * deno
        encolen = jnp.log2(m + 1e-20)
        loss_ref[...] = -jnp.sum((m - q) * encolen, axis=1, keepdims=True)


@jax.jit
def kernel(logits, sigma):
    n, dmod = logits.shape
    k = _NUM_CLUSTERS
    xt = logits.T
    sigma_f = jnp.asarray(sigma, dtype=jnp.float32)
    factor = (1.0 / (2.0 * sigma_f ** 2)).reshape(1, 1)

    ct, c, csum = pl.pallas_call(
        _cluster_body,
        out_shape=(
            jax.ShapeDtypeStruct((k, n), jnp.float32),
            jax.ShapeDtypeStruct((n, k), jnp.float32),
            jax.ShapeDtypeStruct((1, k), jnp.float32),
        ),
    )(logits, xt, factor)

    br = 1024
    nblocks = n // br
    loss = pl.pallas_call(
        _main_body,
        grid=(nblocks,),
        in_specs=[
            pl.BlockSpec((br, dmod), lambda i: (i, 0)),
            pl.BlockSpec((dmod, n), lambda i: (0, 0)),
            pl.BlockSpec((k, n), lambda i: (0, 0)),
            pl.BlockSpec((br, k), lambda i: (i, 0)),
            pl.BlockSpec((1, k), lambda i: (0, 0)),
        ],
        out_specs=pl.BlockSpec((1, 1), lambda i: (0, 0)),
        out_shape=jax.ShapeDtypeStruct((1, 1), jnp.float32),
        scratch_shapes=[
            pltpu.VMEM((1, k), jnp.float32),
            pltpu.VMEM((1, k), jnp.float32),
            pltpu.VMEM((1, 1), jnp.float32),
            pltpu.VMEM((1, 1), jnp.float32),
        ],
    )(logits, xt, ct, c, csum)
    return loss[0, 0]
